# baseline (speedup 1.0000x reference)
import jax
import jax.numpy as jnp
from jax.experimental import pallas as pl
from jax.experimental.pallas import tpu as pltpu

NUM_GRAPHS = 128
HIDDEN = 256
BLOCK = 1024
CH = 64
NCH = BLOCK // CH
NEG_INF = float("-inf")


def _pool_kernel(starts, bi, sj, x_ref, sv_ref, wt_ref, b_ref,
                 o_ref, mx_ref, sm_ref):
    j = pl.program_id(0)
    nw = pl.num_programs(0)

    @pl.when(j == 0)
    def _():
        mx_ref[...] = jnp.full_like(mx_ref, NEG_INF)
        sm_ref[...] = jnp.zeros_like(sm_ref)

    s = sj[j]

    @pl.when(s >= 0)
    def _():
        blk0 = bi[j] * BLOCK
        lo = jnp.maximum(starts[s], blk0) - blk0
        hi = jnp.minimum(starts[s + 1], blk0 + BLOCK) - blk0

        acc_mx = jnp.full((8, HIDDEN), NEG_INF, dtype=jnp.float32)
        acc_sm = jnp.zeros((8, HIDDEN), dtype=jnp.float32)
        rid = jax.lax.broadcasted_iota(jnp.int32, (CH, HIDDEN), 0)
        for k in range(NCH):
            xk = x_ref[k * CH:(k + 1) * CH, :]
            m = (rid >= lo - k * CH) & (rid < hi - k * CH)
            xm = jnp.where(m, xk, NEG_INF).reshape(8, CH // 8, HIDDEN)
            xs = jnp.where(m, xk, 0.0).reshape(8, CH // 8, HIDDEN)
            acc_mx = jnp.maximum(acc_mx, jnp.max(xm, axis=1))
            acc_sm = acc_sm + jnp.sum(xs, axis=1)
        bmax = jnp.max(acc_mx, axis=0, keepdims=True)
        bsum = jnp.sum(acc_sm, axis=0, keepdims=True)
        mx_ref[pl.ds(s, 1), :] = jnp.maximum(mx_ref[pl.ds(s, 1), :], bmax)
        sm_ref[pl.ds(s, 1), :] = sm_ref[pl.ds(s, 1), :] + bsum

    @pl.when(j == nw - 1)
    def _():
        sv = sv_ref[...]
        counts = sv[1:NUM_GRAPHS + 1, :] - sv[:NUM_GRAPHS, :]
        mean = sm_ref[...] / jnp.maximum(counts, 1.0)
        comb = jnp.concatenate([mx_ref[...], mean], axis=1)
        o_ref[...] = jax.lax.dot_general(
            comb, wt_ref[...], (((1,), (0,)), ((), ())),
            preferred_element_type=jnp.float32) + b_ref[...]


@jax.jit
def kernel(x, batch, W, b):
    n, h = x.shape
    batch = batch.astype(jnp.int32)
    nb = pl.cdiv(n, BLOCK)
    npad = nb * BLOCK
    nw = nb + NUM_GRAPHS - 1
    x = jnp.pad(x, ((0, npad - n), (0, 0)))
    segp = jnp.pad(batch, (0, npad - n), constant_values=NUM_GRAPHS)
    firsts = segp[::BLOCK].astype(jnp.int32)
    lasts = jnp.minimum(segp[BLOCK - 1::BLOCK], NUM_GRAPHS - 1
                        ).astype(jnp.int32)
    starts = jnp.searchsorted(batch, jnp.arange(NUM_GRAPHS + 1,
                                                dtype=jnp.int32)
                              ).astype(jnp.int32)
    cnt = lasts - firsts + 1
    offs = jnp.concatenate([jnp.zeros((1,), jnp.int32),
                            jnp.cumsum(cnt)[:-1].astype(jnp.int32)])
    total = offs[-1] + cnt[-1]
    jidx = jnp.arange(nw, dtype=jnp.int32)
    bi = jnp.clip(jnp.searchsorted(offs, jidx, side="right").astype(
        jnp.int32) - 1, 0, nb - 1)
    sj = jnp.where(jidx < total, firsts[bi] + (jidx - offs[bi]), -1
                   ).astype(jnp.int32)
    bi = jnp.where(jidx < total, bi, nb - 1).astype(jnp.int32)
    sv = jnp.pad(starts.astype(jnp.float32),
                 (0, 7)).reshape(NUM_GRAPHS + 8, 1)
    wt = W.T
    b2 = b.reshape(1, h)

    out = pl.pallas_call(
        _pool_kernel,
        grid_spec=pltpu.PrefetchScalarGridSpec(
            num_scalar_prefetch=3,
            grid=(nw,),
            in_specs=[
                pl.BlockSpec((BLOCK, h), lambda j, starts, bi, sj:
                             (bi[j], 0)),
                pl.BlockSpec((NUM_GRAPHS + 8, 1), lambda j, *_: (0, 0)),
                pl.BlockSpec((2 * h, h), lambda j, *_: (0, 0)),
                pl.BlockSpec((1, h), lambda j, *_: (0, 0)),
            ],
            out_specs=pl.BlockSpec((NUM_GRAPHS, h), lambda j, *_: (0, 0)),
            scratch_shapes=[
                pltpu.VMEM((NUM_GRAPHS, h), jnp.float32),
                pltpu.VMEM((NUM_GRAPHS, h), jnp.float32),
            ],
        ),
        out_shape=jax.ShapeDtypeStruct((NUM_GRAPHS, h), jnp.float32),
    )(starts, bi, sj, x, sv, wt, b2)
    return out
